# Initial kernel scaffold; baseline (speedup 1.0000x reference)
import jax
import jax.numpy as jnp
from jax.experimental import pallas as pl


def kernel(x, emb_table):
    raise NotImplementedError("write your pallas kernel here")


# speedup vs baseline: 2.2476x; 2.2476x over previous
import jax
import jax.numpy as jnp
from jax.experimental import pallas as pl


def _concat_body(x_ref, emb_ref, out_ref):
    d = x_ref.shape[-1]
    out_ref[:, :, :d] = x_ref[...]
    bs, batch, _ = x_ref.shape
    emb = emb_ref[...]
    out_ref[:, :, d:] = jnp.broadcast_to(emb[:, None, :], (bs, batch, d))


def kernel(x, emb_table):
    seq, batch, d = x.shape
    d_emb = emb_table.shape[1]
    bs = 256
    grid = (seq // bs,)
    return pl.pallas_call(
        _concat_body,
        grid=grid,
        in_specs=[
            pl.BlockSpec((bs, batch, d), lambda i: (i, 0, 0)),
            pl.BlockSpec((bs, d_emb), lambda i: (i, 0)),
        ],
        out_specs=pl.BlockSpec((bs, batch, d + d_emb), lambda i: (i, 0, 0)),
        out_shape=jax.ShapeDtypeStruct((seq, batch, d + d_emb), x.dtype),
    )(x, emb_table)


# speedup vs baseline: 2.3294x; 1.0364x over previous
import jax
import jax.numpy as jnp
from jax.experimental import pallas as pl


def _concat_body(x_ref, emb_ref, out_ref):
    d = x_ref.shape[-1]
    out_ref[:, :, :d] = x_ref[...]
    bs, batch, _ = x_ref.shape
    emb = emb_ref[...]
    out_ref[:, :, d:] = jnp.broadcast_to(emb[:, None, :], (bs, batch, d))


def kernel(x, emb_table):
    seq, batch, d = x.shape
    d_emb = emb_table.shape[1]
    bs = 512
    grid = (seq // bs,)
    return pl.pallas_call(
        _concat_body,
        grid=grid,
        in_specs=[
            pl.BlockSpec((bs, batch, d), lambda i: (i, 0, 0)),
            pl.BlockSpec((bs, d_emb), lambda i: (i, 0)),
        ],
        out_specs=pl.BlockSpec((bs, batch, d + d_emb), lambda i: (i, 0, 0)),
        out_shape=jax.ShapeDtypeStruct((seq, batch, d + d_emb), x.dtype),
    )(x, emb_table)
